# baseline (speedup 1.0000x reference)
import functools

import numpy as np

import jax
import jax.numpy as jnp
from jax import lax
from jax.experimental import pallas as pl
from jax.experimental.pallas import tpu as pltpu
from jax.experimental.pallas import tpu_sc as plsc

N_NODES = 10000
K_NBR = 16
HEADS = 2
NENC = 6
TSIZE = 4096
FEAT = 2
LEVELS = 8
_B = (1.0e7) ** (1.0 / (LEVELS - 1))
_RES = [float(_B**l) for l in range(LEVELS)]
_P1 = 2654435761
_P2 = 805459861

NW = 32
NPAD = 10240
CPT = NPAD // NW
NCHUNK = 80
CHUNKS = CPT // NCHUNK
LROW = K_NBR * FEAT
LTAB = TSIZE * FEAT
OBWORDS = NENC * NCHUNK * LROW
TILEOUT = LEVELS * CHUNKS * OBWORDS


def _body(xyz_h, edges_h, tab_h, out_h,
          xyz_v, edges_v, relx, rely, relz,
          t0, t1, t2, t3, t4, t5, obuf):
    info = plsc.get_sparse_core_info()
    nc = info.num_cores
    wid = lax.axis_index("s") * nc + lax.axis_index("c")
    n0 = wid * CPT

    iota16 = lax.iota(jnp.int32, 16)
    col0 = iota16 * 2
    col1 = col0 + 1

    pltpu.sync_copy(xyz_h, xyz_v)
    pltpu.sync_copy(edges_h.at[pl.ds(n0 * K_NBR, CPT * K_NBR)], edges_v)

    def rel_body(j, carry):
        erow3 = edges_v[pl.ds(j * 16, 16)] * 3
        ex = plsc.load_gather(xyz_v, [erow3])
        ey = plsc.load_gather(xyz_v, [erow3 + 1])
        ez = plsc.load_gather(xyz_v, [erow3 + 2])
        own = jnp.minimum(n0 + j, N_NODES - 1) * 3
        own3 = jnp.full((16,), own, jnp.int32)
        ox = plsc.load_gather(xyz_v, [own3])
        oy = plsc.load_gather(xyz_v, [own3 + 1])
        oz = plsc.load_gather(xyz_v, [own3 + 2])
        relx[pl.ds(j * 16, 16)] = ox - ex
        rely[pl.ds(j * 16, 16)] = oy - ey
        relz[pl.ds(j * 16, 16)] = oz - ez
        return carry

    lax.fori_loop(0, CPT, rel_body, 0)

    tabs = (t0, t1, t2, t3, t4, t5)

    for l in range(LEVELS):
        for e in range(NENC):
            pltpu.sync_copy(tab_h.at[pl.ds((l * NENC + e) * LTAB, LTAB)], tabs[e])
        res = jnp.float32(_RES[l])

        def node_body(jj, c, par):
            j = c * NCHUNK + jj
            rx = relx[pl.ds(j * 16, 16)]
            ry = rely[pl.ds(j * 16, 16)]
            rz = relz[pl.ds(j * 16, 16)]
            px = rx * res
            py = ry * res
            pz = rz * res
            ix = px.astype(jnp.int32)
            iy = py.astype(jnp.int32)
            iz = pz.astype(jnp.int32)
            fx = px - ix.astype(jnp.float32)
            fy = py - iy.astype(jnp.float32)
            fz = pz - iz.astype(jnp.float32)
            nx = fx < 0.0
            ny = fy < 0.0
            nz = fz < 0.0
            ix = jnp.where(nx, ix - 1, ix)
            iy = jnp.where(ny, iy - 1, iy)
            iz = jnp.where(nz, iz - 1, iz)
            fx = jnp.where(nx, fx + 1.0, fx)
            fy = jnp.where(ny, fy + 1.0, fy)
            fz = jnp.where(nz, fz + 1.0, fz)
            a0 = ix.astype(jnp.uint32)
            a1 = a0 + jnp.uint32(1)
            b0 = iy.astype(jnp.uint32) * jnp.uint32(_P1)
            b1 = b0 + jnp.uint32(_P1)
            c0 = iz.astype(jnp.uint32) * jnp.uint32(_P2)
            c1 = c0 + jnp.uint32(_P2)
            bc = ((b0 ^ c0, b1 ^ c0), (b0 ^ c1, b1 ^ c1))
            gx = 1.0 - fx
            gy = 1.0 - fy
            gz = 1.0 - fz
            wyz = ((gy * gz, fy * gz), (gy * fz, fy * fz))
            idxs = []
            ws = []
            for cz in range(2):
                for cy in range(2):
                    for cx in range(2):
                        hsh = ((a1 if cx else a0) ^ bc[cz][cy]) & jnp.uint32(TSIZE - 1)
                        idxs.append((hsh * jnp.uint32(FEAT)).astype(jnp.int32))
                        ws.append((fx if cx else gx) * wyz[cz][cy])
            rowj = jnp.full((16,), jj * HEADS, jnp.int32)
            for e in range(NENC):
                acc0 = ws[0] * plsc.load_gather(tabs[e], [idxs[0]])
                acc1 = ws[0] * plsc.load_gather(tabs[e], [idxs[0] + 1])
                for q in range(1, 8):
                    acc0 = acc0 + ws[q] * plsc.load_gather(tabs[e], [idxs[q]])
                    acc1 = acc1 + ws[q] * plsc.load_gather(tabs[e], [idxs[q] + 1])
                erow = rowj + ((e // HEADS) * (HEADS * NCHUNK) + e % HEADS)
                plsc.store_scatter(obuf, [erow, col0], acc0)
                plsc.store_scatter(obuf, [erow, col1], acc1)

        def chunk_body(c, carry):
            plsc.parallel_loop(0, NCHUNK)(
                lambda jj, c=c: node_body(jj, c, 0))
            for i in range(3):
                pltpu.sync_copy(
                    obuf.at[pl.ds(i * (HEADS * NCHUNK), HEADS * NCHUNK), :],
                    out_h.at[i, pl.ds((n0 + c * NCHUNK) * HEADS, HEADS * NCHUNK),
                             pl.ds(l * LROW, LROW)])
            return carry

        lax.fori_loop(0, CHUNKS, chunk_body, 0)


def kernel(xyz, edges, tables):
    edges = edges.astype(jnp.int32)
    xyz_f = xyz.reshape(-1)
    edges_p = jnp.pad(edges, ((0, NPAD - N_NODES), (0, 0))).reshape(-1)
    tab_lf = jnp.transpose(tables, (1, 0, 2, 3)).reshape(-1)

    mesh = plsc.VectorSubcoreMesh(core_axis_name="c", subcore_axis_name="s")
    call = functools.partial(
        pl.kernel,
        out_type=jax.ShapeDtypeStruct((3, NPAD * HEADS, LEVELS * LROW),
                                      jnp.float32),
        mesh=mesh,
        compiler_params=pltpu.CompilerParams(needs_layout_passes=False,
                                             use_tc_tiling_on_sc=False),
        scratch_types=[
            pltpu.VMEM((N_NODES * 3,), jnp.float32),
            pltpu.VMEM((CPT * K_NBR,), jnp.int32),
            pltpu.VMEM((CPT * K_NBR,), jnp.float32),
            pltpu.VMEM((CPT * K_NBR,), jnp.float32),
            pltpu.VMEM((CPT * K_NBR,), jnp.float32),
        ] + [pltpu.VMEM((LTAB,), jnp.float32)] * NENC + [
            pltpu.VMEM((NENC * NCHUNK, LROW), jnp.float32),
        ],
    )(_body)
    mid = call(xyz_f, edges_p, tab_lf)
    ncol = K_NBR * LEVELS * FEAT
    perm = np.zeros((ncol, ncol), dtype=np.float32)
    for l in range(LEVELS):
        for k in range(K_NBR):
            for f in range(FEAT):
                perm[l * LROW + k * FEAT + f, k * LEVELS * FEAT + l * FEAT + f] = 1.0
    grid = lax.dot_general(mid[:, :N_NODES * HEADS], jnp.asarray(perm),
                           (((2,), (0,)), ((), ())))
    grid = grid.reshape(3, N_NODES, HEADS, K_NBR, LEVELS * FEAT)
    return (grid[0], grid[1], grid[2])


# speedup vs baseline: 2.1478x; 2.1478x over previous
import functools

import numpy as np

import jax
import jax.numpy as jnp
from jax import lax
from jax.experimental import pallas as pl
from jax.experimental.pallas import tpu as pltpu
from jax.experimental.pallas import tpu_sc as plsc

N_NODES = 10000
K_NBR = 16
HEADS = 2
NENC = 6
TSIZE = 4096
FEAT = 2
LEVELS = 8
_B = (1.0e7) ** (1.0 / (LEVELS - 1))
_RES = [float(_B**l) for l in range(LEVELS)]
_P1 = 2654435761
_P2 = 805459861

NW = 32
NPAD = 10240
CPT = NPAD // NW
NCHUNK = 80
CHUNKS = CPT // NCHUNK
LROW = K_NBR * FEAT
LTAB = TSIZE * FEAT
OBWORDS = NENC * NCHUNK * LROW
TILEOUT = LEVELS * CHUNKS * OBWORDS


def _body(xyz_h, edges_h, tab_h, out_h,
          xyz_v, edges_v, relx, rely, relz,
          t0, t1, t2, t3, t4, t5, obuf):
    info = plsc.get_sparse_core_info()
    nc = info.num_cores
    wid = lax.axis_index("s") * nc + lax.axis_index("c")
    n0 = wid * CPT

    iota16 = lax.iota(jnp.int32, 16)
    col0 = iota16 * 2
    col1 = col0 + 1

    pltpu.sync_copy(xyz_h, xyz_v)
    pltpu.sync_copy(edges_h.at[pl.ds(n0 * K_NBR, CPT * K_NBR)], edges_v)

    def rel_body(j, carry):
        erow3 = edges_v[pl.ds(j * 16, 16)] * 3
        ex = plsc.load_gather(xyz_v, [erow3])
        ey = plsc.load_gather(xyz_v, [erow3 + 1])
        ez = plsc.load_gather(xyz_v, [erow3 + 2])
        own = jnp.minimum(n0 + j, N_NODES - 1) * 3
        own3 = jnp.full((16,), own, jnp.int32)
        ox = plsc.load_gather(xyz_v, [own3])
        oy = plsc.load_gather(xyz_v, [own3 + 1])
        oz = plsc.load_gather(xyz_v, [own3 + 2])
        relx[pl.ds(j * 16, 16)] = ox - ex
        rely[pl.ds(j * 16, 16)] = oy - ey
        relz[pl.ds(j * 16, 16)] = oz - ez
        return carry

    lax.fori_loop(0, CPT, rel_body, 0)

    tabs = (t0, t1, t2, t3, t4, t5)

    for l in range(LEVELS):
        for e in range(NENC):
            pltpu.sync_copy(tab_h.at[pl.ds((l * NENC + e) * LTAB, LTAB)], tabs[e])
        res = jnp.float32(_RES[l])

        def node_body(jj, c, par):
            j = c * NCHUNK + jj
            rx = relx[pl.ds(j * 16, 16)]
            ry = rely[pl.ds(j * 16, 16)]
            rz = relz[pl.ds(j * 16, 16)]
            px = rx * res
            py = ry * res
            pz = rz * res
            ix = px.astype(jnp.int32)
            iy = py.astype(jnp.int32)
            iz = pz.astype(jnp.int32)
            fx = px - ix.astype(jnp.float32)
            fy = py - iy.astype(jnp.float32)
            fz = pz - iz.astype(jnp.float32)
            nx = fx < 0.0
            ny = fy < 0.0
            nz = fz < 0.0
            ix = jnp.where(nx, ix - 1, ix)
            iy = jnp.where(ny, iy - 1, iy)
            iz = jnp.where(nz, iz - 1, iz)
            fx = jnp.where(nx, fx + 1.0, fx)
            fy = jnp.where(ny, fy + 1.0, fy)
            fz = jnp.where(nz, fz + 1.0, fz)
            a0 = ix.astype(jnp.uint32)
            a1 = a0 + jnp.uint32(1)
            b0 = iy.astype(jnp.uint32) * jnp.uint32(_P1)
            b1 = b0 + jnp.uint32(_P1)
            c0 = iz.astype(jnp.uint32) * jnp.uint32(_P2)
            c1 = c0 + jnp.uint32(_P2)
            bc = ((b0 ^ c0, b1 ^ c0), (b0 ^ c1, b1 ^ c1))
            gx = 1.0 - fx
            gy = 1.0 - fy
            gz = 1.0 - fz
            wyz = ((gy * gz, fy * gz), (gy * fz, fy * fz))
            idxs = []
            ws = []
            for cz in range(2):
                for cy in range(2):
                    for cx in range(2):
                        hsh = ((a1 if cx else a0) ^ bc[cz][cy]) & jnp.uint32(TSIZE - 1)
                        idxs.append((hsh * jnp.uint32(FEAT)).astype(jnp.int32))
                        ws.append((fx if cx else gx) * wyz[cz][cy])
            rowj = jnp.full((16,), jj * HEADS, jnp.int32)
            for e in range(NENC):
                acc0 = ws[0] * plsc.load_gather(tabs[e], [idxs[0]])
                acc1 = ws[0] * plsc.load_gather(tabs[e], [idxs[0] + 1])
                for q in range(1, 8):
                    acc0 = acc0 + ws[q] * plsc.load_gather(tabs[e], [idxs[q]])
                    acc1 = acc1 + ws[q] * plsc.load_gather(tabs[e], [idxs[q] + 1])
                erow = rowj + ((e // HEADS) * (HEADS * NCHUNK) + e % HEADS)
                plsc.store_scatter(obuf, [erow, col0], acc0)
                plsc.store_scatter(obuf, [erow, col1], acc1)

        def chunk_body(c, carry):
            plsc.parallel_loop(0, NCHUNK, unroll=4)(
                lambda jj, c=c: node_body(jj, c, 0))
            for i in range(3):
                pltpu.sync_copy(
                    obuf.at[pl.ds(i * (HEADS * NCHUNK), HEADS * NCHUNK), :],
                    out_h.at[i, pl.ds((n0 + c * NCHUNK) * HEADS, HEADS * NCHUNK),
                             pl.ds(l * LROW, LROW)])
            return carry

        lax.fori_loop(0, CHUNKS, chunk_body, 0)


def kernel(xyz, edges, tables):
    edges = edges.astype(jnp.int32)
    xyz_f = xyz.reshape(-1)
    edges_p = jnp.pad(edges, ((0, NPAD - N_NODES), (0, 0))).reshape(-1)
    tab_lf = jnp.transpose(tables, (1, 0, 2, 3)).reshape(-1)

    mesh = plsc.VectorSubcoreMesh(core_axis_name="c", subcore_axis_name="s")
    call = functools.partial(
        pl.kernel,
        out_type=jax.ShapeDtypeStruct((3, NPAD * HEADS, LEVELS * LROW),
                                      jnp.float32),
        mesh=mesh,
        compiler_params=pltpu.CompilerParams(needs_layout_passes=False,
                                             use_tc_tiling_on_sc=False),
        scratch_types=[
            pltpu.VMEM((N_NODES * 3,), jnp.float32),
            pltpu.VMEM((CPT * K_NBR,), jnp.int32),
            pltpu.VMEM((CPT * K_NBR,), jnp.float32),
            pltpu.VMEM((CPT * K_NBR,), jnp.float32),
            pltpu.VMEM((CPT * K_NBR,), jnp.float32),
        ] + [pltpu.VMEM((LTAB,), jnp.float32)] * NENC + [
            pltpu.VMEM((NENC * NCHUNK, LROW), jnp.float32),
        ],
    )(_body)
    mid = call(xyz_f, edges_p, tab_lf)
    ncol = K_NBR * LEVELS * FEAT
    perm = np.zeros((ncol, ncol), dtype=np.float32)
    for l in range(LEVELS):
        for k in range(K_NBR):
            for f in range(FEAT):
                perm[l * LROW + k * FEAT + f, k * LEVELS * FEAT + l * FEAT + f] = 1.0
    grid = lax.dot_general(mid, jnp.asarray(perm),
                           (((2,), (0,)), ((), ())))
    grid = grid.reshape(3, NPAD, HEADS, K_NBR, LEVELS * FEAT)[:, :N_NODES]
    return (grid[0], grid[1], grid[2])
